# baseline (speedup 1.0000x reference)
import jax
import jax.numpy as jnp
from jax.experimental import pallas as pl


def _argmax_body(x_ref, o_ref):
    x = x_ref[...]
    mx = jnp.max(x, axis=1, keepdims=True)
    ii = jax.lax.broadcasted_iota(jnp.int32, x.shape, 1)
    am = jnp.min(jnp.where(x == mx, ii, x.shape[1]), axis=1)
    o_ref[0, 0, :] = am


def _score_body(tk_ref, tl_ref, pk_ref, pj_ref, plog_ref, score_ref, mean_ref):
    tk = tk_ref[...]
    pk = pk_ref[...]
    pj = pj_ref[...]
    tl = tl_ref[...]
    logits = plog_ref[...]
    Bb, L, _ = tk.shape
    D = logits.shape[2]

    mx = jnp.max(logits, axis=2, keepdims=True)
    iD = jax.lax.broadcasted_iota(jnp.int32, logits.shape, 2)
    plen = jnp.min(jnp.where(logits == mx, iD, D), axis=2, keepdims=True)

    iota_k = jax.lax.broadcasted_iota(jnp.int32, (Bb, L, 1), 1)
    iota_j = jax.lax.broadcasted_iota(jnp.int32, (Bb, 1, L), 2)
    m_t_k = iota_k < tl
    m_p_k = iota_k < plen
    m_p_j = iota_j < plen

    eq_tp = (tk == pj) & m_t_k
    c_t = jnp.sum(eq_tp.astype(jnp.int32), axis=1, keepdims=True)
    eq_pp = (pk == pj) & m_p_k & (iota_k < iota_j)
    rank = jnp.sum(eq_pp.astype(jnp.int32), axis=1, keepdims=True)
    matched = (m_p_j & (rank < c_t)).astype(jnp.float32)
    match = jnp.sum(matched, axis=2, keepdims=True)

    len_t = jnp.clip(tl, 0, L).astype(jnp.float32) + 1e-9
    len_p = jnp.clip(plen, 0, L).astype(jnp.float32) + 1e-9
    prec = match / len_p
    len_scale = jnp.exp(jnp.minimum(0.0, 1.0 - len_t / len_p))
    score = jnp.clip(len_scale * jnp.sqrt(prec), 0.0, 1.0)
    w = ((tl > 0) & (plen > 0)).astype(jnp.float32)
    score_ref[...] = score
    mean_ref[...] = score / w


def kernel(y_true_tokens, y_true_length, y_pred_tokens, y_pred_length):
    B, L, NV, V = y_pred_tokens.shape
    D = y_pred_length.shape[1]
    logits2d = y_pred_tokens.reshape(B * L, V)
    R = 512
    nblk = (B * L) // R
    p_flat = pl.pallas_call(
        _argmax_body,
        grid=(nblk,),
        in_specs=[pl.BlockSpec((R, V), lambda i: (i, 0))],
        out_specs=pl.BlockSpec((1, 1, R), lambda i: (i, 0, 0)),
        out_shape=jax.ShapeDtypeStruct((nblk, 1, R), jnp.int32),
    )(logits2d)
    p_k = p_flat.reshape(B, L, 1)
    p_j = p_flat.reshape(B, 1, L)
    t_k = y_true_tokens.reshape(B, L, 1)
    t_l = y_true_length.reshape(B, 1, 1)
    p_log = y_pred_length.reshape(B, 1, D)
    Bb = 64
    score, mean = pl.pallas_call(
        _score_body,
        grid=(B // Bb,),
        in_specs=[pl.BlockSpec((Bb, L, 1), lambda i: (i, 0, 0)),
                  pl.BlockSpec((Bb, 1, 1), lambda i: (i, 0, 0)),
                  pl.BlockSpec((Bb, L, 1), lambda i: (i, 0, 0)),
                  pl.BlockSpec((Bb, 1, L), lambda i: (i, 0, 0)),
                  pl.BlockSpec((Bb, 1, D), lambda i: (i, 0, 0))],
        out_specs=[pl.BlockSpec((Bb, 1, 1), lambda i: (i, 0, 0)),
                   pl.BlockSpec((Bb, 1, 1), lambda i: (i, 0, 0))],
        out_shape=[jax.ShapeDtypeStruct((B, 1, 1), jnp.float32),
                   jax.ShapeDtypeStruct((B, 1, 1), jnp.float32)],
    )(t_k, t_l, p_k, p_j, p_log)
    return (score.reshape(B, NV), mean.reshape(B, NV))


# speedup vs baseline: 1.8687x; 1.8687x over previous
import functools

import jax
import jax.numpy as jnp
from jax import lax
from jax.experimental import pallas as pl
from jax.experimental.pallas import tpu as pltpu
from jax.experimental.pallas import tpu_sc as plsc

_INFO = plsc.get_sparse_core_info()
_NW = _INFO.num_cores * _INFO.num_subcores
_LN = _INFO.num_lanes


def _pad128(n):
    return (n + 127) // 128 * 128


def _argmax_body(x_ref, o_ref):
    x = x_ref[...]
    V = x.shape[2]
    mx = jnp.max(x, axis=2, keepdims=True)
    iV = jax.lax.broadcasted_iota(jnp.int32, x.shape, 2)
    o_ref[...] = jnp.min(jnp.where(x == mx, iV, V), axis=2)


def _sqrt_f32(x):
    y = plsc.bitcast((plsc.bitcast(x, jnp.int32) >> 1) + 0x1FBD1DF5, jnp.float32)
    for _ in range(3):
        y = 0.5 * (y + x / y)
    return y


def _make_sc_match(B, L, D, V):
    E = B // _NW
    G = E // _LN
    mesh = plsc.VectorSubcoreMesh(core_axis_name="c", subcore_axis_name="s")

    @functools.partial(
        pl.kernel,
        out_type=[jax.ShapeDtypeStruct((B,), jnp.float32),
                  jax.ShapeDtypeStruct((B,), jnp.float32)],
        mesh=mesh,
        compiler_params=pltpu.CompilerParams(needs_layout_passes=False),
        scratch_types=[
            pltpu.VMEM((_pad128(E * L),), jnp.int32),
            pltpu.VMEM((_pad128(E * L),), jnp.int32),
            pltpu.VMEM((128,), jnp.int32),
            pltpu.VMEM((_pad128(E * D),), jnp.float32),
            pltpu.VMEM((_LN * V,), jnp.int32),
            pltpu.VMEM((128,), jnp.float32),
            pltpu.VMEM((128,), jnp.float32),
        ],
    )
    def sc_match(t_hbm, p_hbm, tl_hbm, plog_hbm, score_hbm, mean_hbm,
                 t_v, p_v, tl_v, plog_v, hist_v, sc_v, mn_v):
        wid = lax.axis_index("s") * _INFO.num_cores + lax.axis_index("c")
        base = wid * E
        pltpu.sync_copy(t_hbm.at[pl.ds(base * L, E * L)], t_v.at[pl.ds(0, E * L)])
        pltpu.sync_copy(p_hbm.at[pl.ds(base * L, E * L)], p_v.at[pl.ds(0, E * L)])
        pltpu.sync_copy(tl_hbm.at[pl.ds(base, E)], tl_v.at[pl.ds(0, E)])
        pltpu.sync_copy(plog_hbm.at[pl.ds(base * D, E * D)],
                        plog_v.at[pl.ds(0, E * D)])
        lanes = lax.iota(jnp.int32, _LN)
        ones = jnp.ones((_LN,), jnp.int32)

        for g in range(G):
            gl = g * _LN + lanes
            row = gl * L
            hbase = lanes * V

            def zero_body(i, c):
                hist_v[pl.ds(i * _LN, _LN)] = jnp.zeros((_LN,), jnp.int32)
                return c
            lax.fori_loop(0, V, zero_body, 0)

            tl_vec = tl_v[pl.ds(g * _LN, _LN)]

            prow = gl * D

            best = jnp.full((_LN,), float("-inf"), jnp.float32)
            bidx = jnp.zeros((_LN,), jnp.int32)
            for d in range(D):
                v = plsc.load_gather(plog_v, [prow + d])
                upd = v > best
                best = jnp.where(upd, v, best)
                bidx = jnp.where(upd, jnp.full((_LN,), d, jnp.int32), bidx)
            plen_vec = bidx

            for k in range(L):
                tok = plsc.load_gather(t_v, [row + k])
                m = k < tl_vec
                plsc.addupdate_scatter(hist_v, [hbase + tok], ones, mask=m)

            match_vec = jnp.zeros((_LN,), jnp.int32)
            for k in range(L):
                tok = plsc.load_gather(p_v, [row + k])
                m = k < plen_vec
                idx = hbase + tok
                cnt = plsc.load_gather(hist_v, [idx])
                hit = m & (cnt > 0)
                plsc.store_scatter(hist_v, [idx], cnt - 1, mask=hit)
                match_vec = match_vec + jnp.where(hit, 1, 0)

            len_t = jnp.clip(tl_vec, 0, L).astype(jnp.float32) + 1e-9
            len_p = jnp.clip(plen_vec, 0, L).astype(jnp.float32) + 1e-9
            prec = match_vec.astype(jnp.float32) / len_p
            ls = jnp.exp(jnp.minimum(0.0, 1.0 - len_t / len_p))
            score = jnp.clip(ls * _sqrt_f32(prec), 0.0, 1.0)
            w = ((tl_vec > 0) & (plen_vec > 0)).astype(jnp.float32)
            sc_v[pl.ds(g * _LN, _LN)] = score
            mn_v[pl.ds(g * _LN, _LN)] = score / w

        pltpu.sync_copy(sc_v.at[pl.ds(0, E)], score_hbm.at[pl.ds(base, E)])
        pltpu.sync_copy(mn_v.at[pl.ds(0, E)], mean_hbm.at[pl.ds(base, E)])

    return sc_match


def kernel(y_true_tokens, y_true_length, y_pred_tokens, y_pred_length):
    B, L, NV, V = y_pred_tokens.shape
    D = y_pred_length.shape[1]
    x3 = y_pred_tokens.reshape(B, L, V)
    Bb = 16
    p_tok = pl.pallas_call(
        _argmax_body,
        grid=(B // Bb,),
        in_specs=[pl.BlockSpec((Bb, L, V), lambda i: (i, 0, 0))],
        out_specs=pl.BlockSpec((Bb, L), lambda i: (i, 0)),
        out_shape=jax.ShapeDtypeStruct((B, L), jnp.int32),
    )(x3)
    sc = _make_sc_match(B, L, D, V)
    score, mean = sc(y_true_tokens.reshape(B * L),
                     p_tok.reshape(B * L),
                     y_true_length,
                     y_pred_length.reshape(B * D))
    return (score.reshape(B, NV), mean.reshape(B, NV))


# speedup vs baseline: 4.6433x; 2.4849x over previous
import functools

import jax
import jax.numpy as jnp
from jax import lax
from jax.experimental import pallas as pl
from jax.experimental.pallas import tpu as pltpu
from jax.experimental.pallas import tpu_sc as plsc

_INFO = plsc.get_sparse_core_info()
_NW = _INFO.num_cores * _INFO.num_subcores
_LN = _INFO.num_lanes


def _argmax_body(x_ref, o_ref):
    x = x_ref[...]
    V = x.shape[1]
    mx = jnp.max(x, axis=1, keepdims=True)
    iV = jax.lax.broadcasted_iota(jnp.int32, x.shape, 1)
    o_ref[...] = jnp.min(jnp.where(x == mx, iV, V), axis=1, keepdims=True)


def _sqrt_f32(x):
    y = plsc.bitcast((plsc.bitcast(x, jnp.int32) >> 1) + 0x1FBD1DF5, jnp.float32)
    for _ in range(3):
        y = 0.5 * (y + x / y)
    return y


def _make_sc_match(B, L, D, V):
    E = B // _NW
    G = E // _LN
    mesh = plsc.VectorSubcoreMesh(core_axis_name="c", subcore_axis_name="s")

    @functools.partial(
        pl.kernel,
        out_type=[jax.ShapeDtypeStruct((B,), jnp.float32),
                  jax.ShapeDtypeStruct((B,), jnp.float32)],
        mesh=mesh,
        compiler_params=pltpu.CompilerParams(needs_layout_passes=False),
        scratch_types=[
            pltpu.VMEM((L, 128), jnp.int32),
            pltpu.VMEM((L, 128), jnp.int32),
            pltpu.VMEM((128,), jnp.int32),
            pltpu.VMEM((D, 128), jnp.float32),
            pltpu.VMEM((_LN * V,), jnp.int32),
            pltpu.VMEM((128,), jnp.float32),
            pltpu.VMEM((128,), jnp.float32),
        ],
    )
    def sc_match(t_hbm, p_hbm, tl_hbm, plog_hbm, score_hbm, mean_hbm,
                 t_v, p_v, tl_v, plog_v, hist_v, sc_v, mn_v):
        wid = lax.axis_index("s") * _INFO.num_cores + lax.axis_index("c")
        base = wid * E
        tile = base // 128 * 128
        sub = base - tile
        pltpu.sync_copy(t_hbm.at[:, pl.ds(tile, 128)], t_v)
        pltpu.sync_copy(p_hbm.at[:, pl.ds(tile, 128)], p_v)
        pltpu.sync_copy(tl_hbm.at[pl.ds(base, E)], tl_v.at[pl.ds(0, E)])
        pltpu.sync_copy(plog_hbm.at[:, pl.ds(tile, 128)], plog_v)
        lanes = lax.iota(jnp.int32, _LN)
        ones = jnp.ones((_LN,), jnp.int32)

        for g in range(G):
            col = sub + g * _LN + lanes
            hbase = lanes * V

            def zero_body(i, c):
                hist_v[pl.ds(i * _LN, _LN)] = jnp.zeros((_LN,), jnp.int32)
                return c
            lax.fori_loop(0, V, zero_body, 0)

            tl_vec = tl_v[pl.ds(g * _LN, _LN)]

            best = jnp.full((_LN,), float("-inf"), jnp.float32)
            bidx = jnp.zeros((_LN,), jnp.int32)
            for d in range(D):
                v = plsc.load_gather(plog_v, [jnp.full((_LN,), d, jnp.int32), col])
                upd = v > best
                best = jnp.where(upd, v, best)
                bidx = jnp.where(upd, jnp.full((_LN,), d, jnp.int32), bidx)
            plen_vec = bidx

            for k in range(L):
                tok = plsc.load_gather(t_v, [jnp.full((_LN,), k, jnp.int32), col])
                m = k < tl_vec
                plsc.addupdate_scatter(hist_v, [hbase + tok], ones, mask=m)

            match_vec = jnp.zeros((_LN,), jnp.int32)
            for k in range(L):
                tok = plsc.load_gather(p_v, [jnp.full((_LN,), k, jnp.int32), col])
                m = k < plen_vec
                idx = hbase + tok
                cnt = plsc.load_gather(hist_v, [idx])
                hit = m & (cnt > 0)
                plsc.store_scatter(hist_v, [idx], cnt - 1, mask=hit)
                match_vec = match_vec + jnp.where(hit, 1, 0)

            len_t = jnp.clip(tl_vec, 0, L).astype(jnp.float32) + 1e-9
            len_p = jnp.clip(plen_vec, 0, L).astype(jnp.float32) + 1e-9
            prec = match_vec.astype(jnp.float32) / len_p
            ls = jnp.exp(jnp.minimum(0.0, 1.0 - len_t / len_p))
            score = jnp.clip(ls * _sqrt_f32(prec), 0.0, 1.0)
            w = ((tl_vec > 0) & (plen_vec > 0)).astype(jnp.float32)
            sc_v[pl.ds(g * _LN, _LN)] = score
            mn_v[pl.ds(g * _LN, _LN)] = score / w

        pltpu.sync_copy(sc_v.at[pl.ds(0, E)], score_hbm.at[pl.ds(base, E)])
        pltpu.sync_copy(mn_v.at[pl.ds(0, E)], mean_hbm.at[pl.ds(base, E)])

    return sc_match


def kernel(y_true_tokens, y_true_length, y_pred_tokens, y_pred_length):
    B, L, NV, V = y_pred_tokens.shape
    D = y_pred_length.shape[1]
    xT = y_pred_tokens.transpose(1, 2, 3, 0).reshape(L, V, B)
    t2 = y_true_tokens.transpose(1, 2, 0).reshape(L, B)
    plog2 = y_pred_length.transpose(1, 0)
    p_tok = pl.pallas_call(
        _argmax_body,
        grid=(L,),
        in_specs=[pl.BlockSpec((1, V, B), lambda i: (i, 0, 0))],
        out_specs=pl.BlockSpec((1, 1, B), lambda i: (i, 0, 0)),
        out_shape=jax.ShapeDtypeStruct((L, 1, B), jnp.int32),
    )(xT)
    sc = _make_sc_match(B, L, D, V)
    score, mean = sc(t2, p_tok.reshape(L, B), y_true_length, plog2)
    return (score.reshape(B, NV), mean.reshape(B, NV))


# speedup vs baseline: 5.4580x; 1.1755x over previous
import functools

import jax
import jax.numpy as jnp
from jax import lax
from jax.experimental import pallas as pl
from jax.experimental.pallas import tpu as pltpu
from jax.experimental.pallas import tpu_sc as plsc

_INFO = plsc.get_sparse_core_info()
_NW = _INFO.num_cores * _INFO.num_subcores
_LN = _INFO.num_lanes


def _argmax_body(x_ref, o_ref):
    x = x_ref[...]
    V = x.shape[1]
    mx = jnp.max(x, axis=1, keepdims=True)
    iV = jax.lax.broadcasted_iota(jnp.int32, x.shape, 1)
    o_ref[...] = jnp.min(jnp.where(x == mx, iV, V), axis=1, keepdims=True)


def _sqrt_f32(x):
    y = plsc.bitcast((plsc.bitcast(x, jnp.int32) >> 1) + 0x1FBD1DF5, jnp.float32)
    for _ in range(3):
        y = 0.5 * (y + x / y)
    return y


def _make_sc_match(B, L, D, V):
    E = B // _NW
    G = E // _LN
    mesh = plsc.VectorSubcoreMesh(core_axis_name="c", subcore_axis_name="s")

    @functools.partial(
        pl.kernel,
        out_type=[jax.ShapeDtypeStruct((B,), jnp.float32),
                  jax.ShapeDtypeStruct((B,), jnp.float32)],
        mesh=mesh,
        compiler_params=pltpu.CompilerParams(needs_layout_passes=False),
        scratch_types=[
            pltpu.VMEM((L, 128), jnp.int32),
            pltpu.VMEM((L, 128), jnp.int32),
            pltpu.VMEM((128,), jnp.int32),
            pltpu.VMEM((D, 128), jnp.float32),
            pltpu.VMEM((_LN * V,), jnp.int32),
            pltpu.VMEM((128,), jnp.float32),
            pltpu.VMEM((128,), jnp.float32),
        ],
    )
    def sc_match(t_hbm, p_hbm, tl_hbm, plog_hbm, z_hbm, score_hbm, mean_hbm,
                 t_v, p_v, tl_v, plog_v, hist_v, sc_v, mn_v):
        wid = lax.axis_index("s") * _INFO.num_cores + lax.axis_index("c")
        base = wid * E
        tile = base // 128 * 128
        sub = base - tile
        pltpu.sync_copy(t_hbm.at[:, pl.ds(tile, 128)], t_v)
        pltpu.sync_copy(p_hbm.at[:, pl.ds(tile, 128)], p_v)
        pltpu.sync_copy(tl_hbm.at[pl.ds(base, E)], tl_v.at[pl.ds(0, E)])
        pltpu.sync_copy(plog_hbm.at[:, pl.ds(tile, 128)], plog_v)
        lanes = lax.iota(jnp.int32, _LN)
        ones = jnp.ones((_LN,), jnp.int32)

        for g in range(G):
            col = sub + g * _LN + lanes
            hbase = lanes * V

            pltpu.sync_copy(z_hbm, hist_v)

            tl_vec = tl_v[pl.ds(g * _LN, _LN)]

            best = jnp.full((_LN,), float("-inf"), jnp.float32)
            bidx = jnp.zeros((_LN,), jnp.int32)
            for d in range(D):
                v = plsc.load_gather(plog_v, [jnp.full((_LN,), d, jnp.int32), col])
                upd = v > best
                best = jnp.where(upd, v, best)
                bidx = jnp.where(upd, jnp.full((_LN,), d, jnp.int32), bidx)
            plen_vec = bidx

            for k in range(L):
                tok = plsc.load_gather(t_v, [jnp.full((_LN,), k, jnp.int32), col])
                m = k < tl_vec
                plsc.addupdate_scatter(hist_v, [hbase + tok], ones, mask=m)

            match_vec = jnp.zeros((_LN,), jnp.int32)
            for k in range(L):
                tok = plsc.load_gather(p_v, [jnp.full((_LN,), k, jnp.int32), col])
                m = k < plen_vec
                idx = hbase + tok
                cnt = plsc.load_gather(hist_v, [idx])
                hit = m & (cnt > 0)
                plsc.store_scatter(hist_v, [idx], cnt - 1, mask=hit)
                match_vec = match_vec + jnp.where(hit, 1, 0)

            len_t = jnp.clip(tl_vec, 0, L).astype(jnp.float32) + 1e-9
            len_p = jnp.clip(plen_vec, 0, L).astype(jnp.float32) + 1e-9
            prec = match_vec.astype(jnp.float32) / len_p
            ls = jnp.exp(jnp.minimum(0.0, 1.0 - len_t / len_p))
            score = jnp.clip(ls * _sqrt_f32(prec), 0.0, 1.0)
            w = ((tl_vec > 0) & (plen_vec > 0)).astype(jnp.float32)
            sc_v[pl.ds(g * _LN, _LN)] = score
            mn_v[pl.ds(g * _LN, _LN)] = score / w

        pltpu.sync_copy(sc_v.at[pl.ds(0, E)], score_hbm.at[pl.ds(base, E)])
        pltpu.sync_copy(mn_v.at[pl.ds(0, E)], mean_hbm.at[pl.ds(base, E)])

    return sc_match


def kernel(y_true_tokens, y_true_length, y_pred_tokens, y_pred_length):
    B, L, NV, V = y_pred_tokens.shape
    D = y_pred_length.shape[1]
    xT = y_pred_tokens.transpose(1, 2, 3, 0).reshape(L, V, B)
    t2 = y_true_tokens.transpose(1, 2, 0).reshape(L, B)
    plog2 = y_pred_length.transpose(1, 0)
    Lb = 2
    p_tok = pl.pallas_call(
        _argmax_body,
        grid=(L // Lb,),
        in_specs=[pl.BlockSpec((Lb, V, B), lambda i: (i, 0, 0))],
        out_specs=pl.BlockSpec((Lb, 1, B), lambda i: (i, 0, 0)),
        out_shape=jax.ShapeDtypeStruct((L, 1, B), jnp.int32),
    )(xT)
    zeros_hist = jnp.zeros((_LN * V,), jnp.int32)
    sc = _make_sc_match(B, L, D, V)
    score, mean = sc(t2, p_tok.reshape(L, B), y_true_length, plog2, zeros_hist)
    return (score.reshape(B, NV), mean.reshape(B, NV))


# speedup vs baseline: 5.5215x; 1.0116x over previous
import functools

import jax
import jax.numpy as jnp
from jax import lax
from jax.experimental import pallas as pl
from jax.experimental.pallas import tpu as pltpu
from jax.experimental.pallas import tpu_sc as plsc

_INFO = plsc.get_sparse_core_info()
_NW = _INFO.num_cores * _INFO.num_subcores
_LN = _INFO.num_lanes


def _argmax_body(x_ref, o_ref):
    x = x_ref[...]
    V = x.shape[1]
    mx = jnp.max(x, axis=1, keepdims=True)
    iV = jax.lax.broadcasted_iota(jnp.int32, x.shape, 1)
    o_ref[...] = jnp.min(jnp.where(x == mx, iV, V), axis=1, keepdims=True)


def _sqrt_f32(x):
    y = plsc.bitcast((plsc.bitcast(x, jnp.int32) >> 1) + 0x1FBD1DF5, jnp.float32)
    for _ in range(3):
        y = 0.5 * (y + x / y)
    return y


def _make_sc_match(B, L, D, V):
    E = B // _NW
    G = E // _LN
    mesh = plsc.VectorSubcoreMesh(core_axis_name="c", subcore_axis_name="s")

    @functools.partial(
        pl.kernel,
        out_type=[jax.ShapeDtypeStruct((B,), jnp.float32),
                  jax.ShapeDtypeStruct((B,), jnp.float32)],
        mesh=mesh,
        compiler_params=pltpu.CompilerParams(needs_layout_passes=False),
        scratch_types=[
            pltpu.VMEM((L, 128), jnp.int32),
            pltpu.VMEM((L, 128), jnp.int32),
            pltpu.VMEM((128,), jnp.int32),
            pltpu.VMEM((D, 128), jnp.float32),
            pltpu.VMEM((_LN * V,), jnp.int32),
            pltpu.VMEM((128,), jnp.float32),
            pltpu.VMEM((128,), jnp.float32),
        ],
    )
    def sc_match(t_hbm, p_hbm, tl_hbm, plog_hbm, z_hbm, score_hbm, mean_hbm,
                 t_v, p_v, tl_v, plog_v, hist_v, sc_v, mn_v):
        wid = lax.axis_index("s") * _INFO.num_cores + lax.axis_index("c")
        base = wid * E
        tile = base // 128 * 128
        sub = base - tile
        pltpu.sync_copy(t_hbm.at[:, pl.ds(tile, 128)], t_v)
        pltpu.sync_copy(p_hbm.at[:, pl.ds(tile, 128)], p_v)
        pltpu.sync_copy(tl_hbm.at[pl.ds(base, E)], tl_v.at[pl.ds(0, E)])
        pltpu.sync_copy(plog_hbm.at[:, pl.ds(tile, 128)], plog_v)
        lanes = lax.iota(jnp.int32, _LN)
        ones = jnp.ones((_LN,), jnp.int32)

        for g in range(G):
            col = sub + g * _LN + lanes
            hbase = lanes * V

            pltpu.sync_copy(z_hbm, hist_v)

            tl_vec = tl_v[pl.ds(g * _LN, _LN)]

            best = jnp.full((_LN,), float("-inf"), jnp.float32)
            bidx = jnp.zeros((_LN,), jnp.int32)
            for d in range(D):
                v = plsc.load_gather(plog_v, [jnp.full((_LN,), d, jnp.int32), col])
                upd = v > best
                best = jnp.where(upd, v, best)
                bidx = jnp.where(upd, jnp.full((_LN,), d, jnp.int32), bidx)
            plen_vec = bidx

            for k in range(L):
                tok = plsc.load_gather(t_v, [jnp.full((_LN,), k, jnp.int32), col])
                m = k < tl_vec
                plsc.addupdate_scatter(hist_v, [hbase + tok], ones, mask=m)

            match_vec = jnp.zeros((_LN,), jnp.int32)
            for k in range(L):
                tok = plsc.load_gather(p_v, [jnp.full((_LN,), k, jnp.int32), col])
                m = k < plen_vec
                idx = hbase + tok
                cnt = plsc.load_gather(hist_v, [idx])
                hit = m & (cnt > 0)
                plsc.store_scatter(hist_v, [idx], cnt - 1, mask=hit)
                match_vec = match_vec + jnp.where(hit, 1, 0)

            len_t = jnp.clip(tl_vec, 0, L).astype(jnp.float32) + 1e-9
            len_p = jnp.clip(plen_vec, 0, L).astype(jnp.float32) + 1e-9
            prec = match_vec.astype(jnp.float32) / len_p
            ls = jnp.exp(jnp.minimum(0.0, 1.0 - len_t / len_p))
            score = jnp.clip(ls * _sqrt_f32(prec), 0.0, 1.0)
            w = ((tl_vec > 0) & (plen_vec > 0)).astype(jnp.float32)
            sc_v[pl.ds(g * _LN, _LN)] = score
            mn_v[pl.ds(g * _LN, _LN)] = score / w

        pltpu.sync_copy(sc_v.at[pl.ds(0, E)], score_hbm.at[pl.ds(base, E)])
        pltpu.sync_copy(mn_v.at[pl.ds(0, E)], mean_hbm.at[pl.ds(base, E)])

    return sc_match


def kernel(y_true_tokens, y_true_length, y_pred_tokens, y_pred_length):
    B, L, NV, V = y_pred_tokens.shape
    D = y_pred_length.shape[1]
    xT = y_pred_tokens.transpose(1, 2, 3, 0).reshape(L, V, B)
    t2 = y_true_tokens.transpose(1, 2, 0).reshape(L, B)
    plog2 = y_pred_length.transpose(1, 0)
    Lb = 5
    p_tok = pl.pallas_call(
        _argmax_body,
        grid=(L // Lb,),
        in_specs=[pl.BlockSpec((Lb, V, B), lambda i: (i, 0, 0))],
        out_specs=pl.BlockSpec((Lb, 1, B), lambda i: (i, 0, 0)),
        out_shape=jax.ShapeDtypeStruct((L, 1, B), jnp.int32),
    )(xT)
    zeros_hist = jnp.zeros((_LN * V,), jnp.int32)
    sc = _make_sc_match(B, L, D, V)
    score, mean = sc(t2, p_tok.reshape(L, B), y_true_length, plog2, zeros_hist)
    return (score.reshape(B, NV), mean.reshape(B, NV))
